# Initial kernel scaffold; baseline (speedup 1.0000x reference)
import jax
import jax.numpy as jnp
from jax.experimental import pallas as pl


def kernel(x, edge_idx, lin_weight, att_dst, att_src, bias):
    raise NotImplementedError("write your pallas kernel here")


# speedup vs baseline: 21.0230x; 21.0230x over previous
import functools

import jax
import jax.numpy as jnp
from jax import lax
from jax.experimental import pallas as pl
from jax.experimental.pallas import tpu as pltpu
from jax.experimental.pallas import tpu_sc as plsc

_N = 10000
_E = 320000
_D = 128
_CP = 144
_NC = 2
_NS = 16
_NW = _NC * _NS
_EPW = _E // _NW
_K = 80
_NCHUNK = _EPW // _K
_NP = 10240
_RPT = _NP // _NS
_LANES = 16


def _proj_body(x_ref, w_ref, as_ref, ad_ref, hext_ref, av_ref, dv_ref):
    x = x_ref[...]
    h = lax.dot_general(x, w_ref[...], (((1,), (1,)), ((), ())),
                        preferred_element_type=jnp.float32)
    b = h.shape[0]
    tail = (lax.broadcasted_iota(jnp.int32, (b, _CP - _D), 1) == 0)
    hext_ref[...] = jnp.concatenate([h, tail.astype(jnp.float32)], axis=1)
    av_ref[...] = jnp.sum(h * as_ref[...], axis=1, keepdims=True)
    dv_ref[...] = jnp.sum(h * ad_ref[...], axis=1, keepdims=True)


def _edge_body(hext_hbm, asrc_hbm, adst_hbm, src_hbm, dst_hbm, zero_hbm,
               outp_hbm, asrc_v, adst_v, src_c, dst_c, rows, wbuf, acc, sem):
    cid = lax.axis_index("c")
    sid = lax.axis_index("s")
    wid = cid * _NS + sid
    pltpu.sync_copy(zero_hbm, acc.at[pl.ds(sid * _RPT, _RPT)])
    pltpu.sync_copy(asrc_hbm, asrc_v)
    pltpu.sync_copy(adst_hbm, adst_v)
    plsc.subcore_barrier()

    base = wid * _EPW

    def chunk(j, carry):
        off = base + j * _K
        pltpu.sync_copy(src_hbm.at[pl.ds(off, _K)], src_c)
        pltpu.sync_copy(dst_hbm.at[pl.ds(off, _K)], dst_c)
        pltpu.async_copy(hext_hbm.at[src_c], rows, sem).wait()
        for g in range(_K // _LANES):
            sv = src_c[pl.ds(g * _LANES, _LANES)]
            dv = dst_c[pl.ds(g * _LANES, _LANES)]
            a = plsc.load_gather(asrc_v, [sv]) + plsc.load_gather(adst_v, [dv])
            a = jnp.where(a >= 0, a, 0.2 * a)
            wbuf[pl.ds(g * _LANES, _LANES)] = jnp.exp(a)

        def scale(e, c2):
            w16 = plsc.load_gather(wbuf, [jnp.full((_LANES,), e, jnp.int32)])
            for v in range(_CP // _LANES):
                sl = pl.ds(v * _LANES, _LANES)
                rows[e, sl] = rows[e, sl] * w16
            return c2

        lax.fori_loop(0, _K, scale, 0)
        pltpu.sync_copy(rows, acc.at[dst_c], add=True)
        return carry

    lax.fori_loop(0, _NCHUNK, chunk, 0)
    plsc.subcore_barrier()
    pltpu.sync_copy(acc.at[pl.ds(sid * _RPT, _RPT)],
                    outp_hbm.at[cid, pl.ds(sid * _RPT, _RPT)])


def _combine_body(p_ref, b_ref, o_ref):
    s = p_ref[0] + p_ref[1]
    num = s[:, :_D]
    den = s[:, _D:_D + 1]
    o_ref[...] = num / (den + 1e-16) + b_ref[...]


def kernel(x, edge_idx, lin_weight, att_dst, att_src, bias):
    n, d = x.shape
    hc = lin_weight.shape[0]
    assert n == _N and d == _D and hc == _D and edge_idx.shape == (2, _E)

    asr = att_src.reshape(1, hc).astype(jnp.float32)
    adt = att_dst.reshape(1, hc).astype(jnp.float32)

    b1 = 1000
    hext, a_src, a_dst = pl.pallas_call(
        _proj_body,
        grid=(n // b1,),
        in_specs=[
            pl.BlockSpec((b1, d), lambda i: (i, 0)),
            pl.BlockSpec((hc, d), lambda i: (0, 0)),
            pl.BlockSpec((1, hc), lambda i: (0, 0)),
            pl.BlockSpec((1, hc), lambda i: (0, 0)),
        ],
        out_specs=[
            pl.BlockSpec((b1, _CP), lambda i: (i, 0)),
            pl.BlockSpec((b1, 1), lambda i: (i, 0)),
            pl.BlockSpec((b1, 1), lambda i: (i, 0)),
        ],
        out_shape=[
            jax.ShapeDtypeStruct((n, _CP), jnp.float32),
            jax.ShapeDtypeStruct((n, 1), jnp.float32),
            jax.ShapeDtypeStruct((n, 1), jnp.float32),
        ],
    )(x, lin_weight, asr, adt)
    a_src = a_src.reshape(n)
    a_dst = a_dst.reshape(n)

    src = edge_idx[0]
    dst = edge_idx[1]
    zeros = jnp.zeros((_RPT, _CP), jnp.float32)

    mesh = plsc.VectorSubcoreMesh(core_axis_name="c", subcore_axis_name="s")
    edge_kernel = functools.partial(
        pl.kernel,
        out_type=jax.ShapeDtypeStruct((_NC, _NP, _CP), jnp.float32),
        mesh=mesh,
        compiler_params=pltpu.CompilerParams(
            needs_layout_passes=False, use_tc_tiling_on_sc=False),
        scratch_types=[
            pltpu.VMEM((_N,), jnp.float32),
            pltpu.VMEM((_N,), jnp.float32),
            pltpu.VMEM((_K,), jnp.int32),
            pltpu.VMEM((_K,), jnp.int32),
            pltpu.VMEM((_K, _CP), jnp.float32),
            pltpu.VMEM((_K,), jnp.float32),
            pltpu.VMEM_SHARED((_NP, _CP), jnp.float32),
            pltpu.SemaphoreType.DMA,
        ],
    )(_edge_body)
    outp = edge_kernel(hext, a_src, a_dst, src, dst, zeros)

    b2 = 1000
    out = pl.pallas_call(
        _combine_body,
        grid=(n // b2,),
        in_specs=[
            pl.BlockSpec((_NC, b2, _CP), lambda i: (0, i, 0)),
            pl.BlockSpec((1, hc), lambda i: (0, 0)),
        ],
        out_specs=pl.BlockSpec((b2, hc), lambda i: (i, 0)),
        out_shape=jax.ShapeDtypeStruct((n, hc), jnp.float32),
    )(outp, bias.reshape(1, hc))
    return out


# speedup vs baseline: 29.0926x; 1.3838x over previous
import functools

import jax
import jax.numpy as jnp
from jax import lax
from jax.experimental import pallas as pl
from jax.experimental.pallas import tpu as pltpu
from jax.experimental.pallas import tpu_sc as plsc

_N = 10000
_E = 320000
_D = 128
_CP = 144
_NC = 2
_NS = 16
_NW = _NC * _NS
_EPW = _E // _NW
_K = 80
_NCHUNK = _EPW // _K
_NPAIR = (_NCHUNK - 1) // 2
_SCALE_UNROLL = 4
_NP = 10240
_RPT = _NP // _NS
_LANES = 16


def _proj_body(x_ref, w_ref, as_ref, ad_ref, hext_ref, av_ref, dv_ref):
    x = x_ref[...]
    h = lax.dot_general(x, w_ref[...], (((1,), (1,)), ((), ())),
                        preferred_element_type=jnp.float32)
    b = h.shape[0]
    tail = (lax.broadcasted_iota(jnp.int32, (b, _CP - _D), 1) == 0)
    hext_ref[...] = jnp.concatenate([h, tail.astype(jnp.float32)], axis=1)
    av_ref[...] = jnp.sum(h * as_ref[...], axis=1, keepdims=True)
    dv_ref[...] = jnp.sum(h * ad_ref[...], axis=1, keepdims=True)


def _edge_body(hext_hbm, asrc_hbm, adst_hbm, src_hbm, dst_hbm, zero_hbm,
               outp_hbm, src_a, dst_a, av_a, bv_a, rows_a, src_b, dst_b, av_b,
               bv_b, rows_b, wbuf, acc, sem_ra, sem_va, sem_rb, sem_vb):
    cid = lax.axis_index("c")
    sid = lax.axis_index("s")
    wid = cid * _NS + sid
    pltpu.sync_copy(zero_hbm, acc.at[pl.ds(sid * _RPT, _RPT)])
    plsc.subcore_barrier()

    base = wid * _EPW

    def fetch(j, src_c, dst_c, av_c, bv_c, rows, sem_r, sem_v):
        off = base + j * _K
        pltpu.sync_copy(src_hbm.at[pl.ds(off, _K)], src_c)
        pltpu.sync_copy(dst_hbm.at[pl.ds(off, _K)], dst_c)
        pltpu.async_copy(hext_hbm.at[src_c], rows, sem_r)
        pltpu.async_copy(asrc_hbm.at[src_c], av_c, sem_v)
        pltpu.async_copy(adst_hbm.at[dst_c], bv_c, sem_v)

    def process(src_c, dst_c, av_c, bv_c, rows, sem_r, sem_v):
        pltpu.make_async_copy(asrc_hbm.at[src_c], av_c, sem_v).wait()
        pltpu.make_async_copy(adst_hbm.at[dst_c], bv_c, sem_v).wait()
        for g in range(_K // _LANES):
            sl = pl.ds(g * _LANES, _LANES)
            a = av_c[sl] + bv_c[sl]
            a = jnp.where(a >= 0, a, 0.2 * a)
            wbuf[sl] = jnp.exp(a)
        pltpu.make_async_copy(hext_hbm.at[src_c], rows, sem_r).wait()

        def scale(i, c2):
            for u in range(_SCALE_UNROLL):
                e = i * _SCALE_UNROLL + u
                w16 = plsc.load_gather(
                    wbuf, [jnp.full((_LANES,), e, jnp.int32)])
                for v in range(_CP // _LANES):
                    sl = pl.ds(v * _LANES, _LANES)
                    rows[e, sl] = rows[e, sl] * w16
            return c2

        lax.fori_loop(0, _K // _SCALE_UNROLL, scale, 0)
        pltpu.sync_copy(rows, acc.at[dst_c], add=True)

    fetch(0, src_a, dst_a, av_a, bv_a, rows_a, sem_ra, sem_va)

    def pair(p, carry):
        j0 = 2 * p
        fetch(j0 + 1, src_b, dst_b, av_b, bv_b, rows_b, sem_rb, sem_vb)
        process(src_a, dst_a, av_a, bv_a, rows_a, sem_ra, sem_va)
        fetch(j0 + 2, src_a, dst_a, av_a, bv_a, rows_a, sem_ra, sem_va)
        process(src_b, dst_b, av_b, bv_b, rows_b, sem_rb, sem_vb)
        return carry

    lax.fori_loop(0, _NPAIR, pair, 0)
    process(src_a, dst_a, av_a, bv_a, rows_a, sem_ra, sem_va)

    plsc.subcore_barrier()
    pltpu.sync_copy(acc.at[pl.ds(sid * _RPT, _RPT)],
                    outp_hbm.at[cid, pl.ds(sid * _RPT, _RPT)])


def _combine_body(p_ref, b_ref, o_ref):
    s = p_ref[0] + p_ref[1]
    num = s[:, :_D]
    den = s[:, _D:_D + 1]
    o_ref[...] = num / (den + 1e-16) + b_ref[...]


def kernel(x, edge_idx, lin_weight, att_dst, att_src, bias):
    n, d = x.shape
    hc = lin_weight.shape[0]
    assert n == _N and d == _D and hc == _D and edge_idx.shape == (2, _E)

    asr = att_src.reshape(1, hc).astype(jnp.float32)
    adt = att_dst.reshape(1, hc).astype(jnp.float32)

    b1 = 1000
    hext, a_src, a_dst = pl.pallas_call(
        _proj_body,
        grid=(n // b1,),
        in_specs=[
            pl.BlockSpec((b1, d), lambda i: (i, 0)),
            pl.BlockSpec((hc, d), lambda i: (0, 0)),
            pl.BlockSpec((1, hc), lambda i: (0, 0)),
            pl.BlockSpec((1, hc), lambda i: (0, 0)),
        ],
        out_specs=[
            pl.BlockSpec((b1, _CP), lambda i: (i, 0)),
            pl.BlockSpec((b1, 1), lambda i: (i, 0)),
            pl.BlockSpec((b1, 1), lambda i: (i, 0)),
        ],
        out_shape=[
            jax.ShapeDtypeStruct((n, _CP), jnp.float32),
            jax.ShapeDtypeStruct((n, 1), jnp.float32),
            jax.ShapeDtypeStruct((n, 1), jnp.float32),
        ],
    )(x, lin_weight, asr, adt)
    a_src = a_src.reshape(n)
    a_dst = a_dst.reshape(n)

    src = edge_idx[0]
    dst = edge_idx[1]
    zeros = jnp.zeros((_RPT, _CP), jnp.float32)

    mesh = plsc.VectorSubcoreMesh(core_axis_name="c", subcore_axis_name="s")
    edge_kernel = functools.partial(
        pl.kernel,
        out_type=jax.ShapeDtypeStruct((_NC, _NP, _CP), jnp.float32),
        mesh=mesh,
        compiler_params=pltpu.CompilerParams(
            needs_layout_passes=False, use_tc_tiling_on_sc=False),
        scratch_types=[
            pltpu.VMEM((_K,), jnp.int32),
            pltpu.VMEM((_K,), jnp.int32),
            pltpu.VMEM((_K,), jnp.float32),
            pltpu.VMEM((_K,), jnp.float32),
            pltpu.VMEM((_K, _CP), jnp.float32),
            pltpu.VMEM((_K,), jnp.int32),
            pltpu.VMEM((_K,), jnp.int32),
            pltpu.VMEM((_K,), jnp.float32),
            pltpu.VMEM((_K,), jnp.float32),
            pltpu.VMEM((_K, _CP), jnp.float32),
            pltpu.VMEM((_K,), jnp.float32),
            pltpu.VMEM_SHARED((_NP, _CP), jnp.float32),
            pltpu.SemaphoreType.DMA,
            pltpu.SemaphoreType.DMA,
            pltpu.SemaphoreType.DMA,
            pltpu.SemaphoreType.DMA,
        ],
    )(_edge_body)
    outp = edge_kernel(hext, a_src, a_dst, src, dst, zeros)

    b2 = 1000
    out = pl.pallas_call(
        _combine_body,
        grid=(n // b2,),
        in_specs=[
            pl.BlockSpec((_NC, b2, _CP), lambda i: (0, i, 0)),
            pl.BlockSpec((1, hc), lambda i: (0, 0)),
        ],
        out_specs=pl.BlockSpec((b2, hc), lambda i: (i, 0)),
        out_shape=jax.ShapeDtypeStruct((n, hc), jnp.float32),
    )(outp, bias.reshape(1, hc))
    return out
